# baseline (speedup 1.0000x reference)
import functools

import jax
import jax.numpy as jnp
from jax import lax
from jax.experimental import pallas as pl
from jax.experimental.pallas import tpu as pltpu
from jax.experimental.pallas import tpu_sc as plsc

_B, _S, _D = 16, 2048, 1024
_R = _B * _S
_NW = 32
_RPW = _R // _NW
_CH = 32
_NC = _RPW // _CH


def kernel(x, lens):
    x2 = x.reshape(_R, _D)
    mesh = plsc.VectorSubcoreMesh(core_axis_name="c", subcore_axis_name="s")

    @functools.partial(
        pl.kernel,
        mesh=mesh,
        out_type=jax.ShapeDtypeStruct((_R, _D), jnp.float32),
        scratch_types=[
            pltpu.VMEM((16,), jnp.int32),
            pltpu.VMEM((1, _D), jnp.float32),
            pltpu.VMEM((_CH, _D), jnp.float32),
            pltpu.VMEM((_CH, _D), jnp.float32),
        ],
    )
    def body(x_hbm, lens_hbm, out_hbm, lens_v, fill1_v, fill_v, stage_v):
        cid = lax.axis_index("c")
        sid = lax.axis_index("s")
        wid = sid * 2 + cid
        base = wid * _RPW
        b = wid // 2

        pltpu.sync_copy(lens_hbm, lens_v)
        lv = lens_v[...]
        L = lax.switch(b, [lambda i=i: lv[i] for i in range(_B)])
        c = jnp.clip(L - (wid % 2) * _RPW, 0, _RPW)

        neg = jnp.full((16,), jnp.float32(-10000.0))

        def fset(i, carry):
            fill1_v[0, pl.ds(i * 16, 16)] = neg
            return carry

        lax.fori_loop(0, _D // 16, fset, 0)

        def frow(i, carry):
            fill_v[i // (_D // 16), pl.ds((i % (_D // 16)) * 16, 16)] = neg
            return carry

        lax.fori_loop(0, _CH * (_D // 16), frow, 0)

        ncopy = (c + _CH - 1) // _CH

        def copy_chunk(i, carry):
            s = base + i * _CH
            pltpu.sync_copy(x_hbm.at[pl.ds(s, _CH)], stage_v)
            pltpu.sync_copy(stage_v, out_hbm.at[pl.ds(s, _CH)])
            return carry

        lax.fori_loop(0, ncopy, copy_chunk, 0)

        def row_fill(r, carry):
            pltpu.sync_copy(fill1_v, out_hbm.at[pl.ds(base + r, 1)])
            return carry

        lax.fori_loop(c, jnp.minimum(ncopy * _CH, _RPW), row_fill, 0)

        def fill_chunk(i, carry):
            s = base + i * _CH
            pltpu.sync_copy(fill_v, out_hbm.at[pl.ds(s, _CH)])
            return carry

        lax.fori_loop(ncopy, _NC, fill_chunk, 0)

    out = body(x2, lens.astype(jnp.int32))
    return out.reshape(_B, _S, _D)


# speedup vs baseline: 1.3016x; 1.3016x over previous
import functools

import jax
import jax.numpy as jnp
from jax import lax
from jax.experimental import pallas as pl
from jax.experimental.pallas import tpu as pltpu
from jax.experimental.pallas import tpu_sc as plsc

_B, _S, _D = 16, 2048, 1024
_R = _B * _S
_NW = 32
_RPW = _R // _NW
_CH = 32
_NC = _RPW // _CH


def kernel(x, lens):
    x2 = x.reshape(_R, _D)
    mesh = plsc.VectorSubcoreMesh(core_axis_name="c", subcore_axis_name="s")

    @functools.partial(
        pl.kernel,
        mesh=mesh,
        out_type=jax.ShapeDtypeStruct((_R, _D), jnp.float32),
        scratch_types=[
            pltpu.VMEM((16,), jnp.int32),
            pltpu.VMEM((1, _D), jnp.float32),
            pltpu.VMEM((_CH, _D), jnp.float32),
            pltpu.VMEM((2, _CH, _D), jnp.float32),
            pltpu.SemaphoreType.DMA((2,)),
            pltpu.SemaphoreType.DMA((2,)),
            pltpu.SemaphoreType.DMA,
        ],
    )
    def body(x_hbm, lens_hbm, out_hbm, lens_v, fill1_v, fill_v, stage_v,
             rsem, wsem, fsem):
        cid = lax.axis_index("c")
        sid = lax.axis_index("s")
        wid = sid * 2 + cid
        base = wid * _RPW
        b = wid // 2

        pltpu.sync_copy(lens_hbm, lens_v)
        lv = lens_v[...]
        L = lax.switch(b, [lambda i=i: lv[i] for i in range(_B)])
        c = jnp.clip(L - (wid % 2) * _RPW, 0, _RPW)
        ncopy = (c + _CH - 1) // _CH

        neg = jnp.full((16,), jnp.float32(-10000.0))

        def fset(i, carry):
            fill1_v[0, pl.ds(i * 16, 16)] = neg
            return carry

        lax.fori_loop(0, _D // 16, fset, 0)

        def frow(i, carry):
            fill_v[i // (_D // 16), pl.ds((i % (_D // 16)) * 16, 16)] = neg
            return carry

        lax.fori_loop(0, _CH * (_D // 16), frow, 0)

        def _fill_copy(i):
            s = base + i * _CH
            return pltpu.make_async_copy(fill_v, out_hbm.at[pl.ds(s, _CH)], fsem)

        def _read(i, sl):
            s = base + i * _CH
            return pltpu.make_async_copy(
                x_hbm.at[pl.ds(s, _CH)], stage_v.at[sl], rsem.at[sl])

        def _write(i, sl):
            s = base + i * _CH
            return pltpu.make_async_copy(
                stage_v.at[sl], out_hbm.at[pl.ds(s, _CH)], wsem.at[sl])

        def fill_start(i, carry):
            _fill_copy(i).start()
            return carry

        lax.fori_loop(ncopy, _NC, fill_start, 0)

        @pl.when(ncopy > 0)
        def _():
            _read(0, 0).start()

        def copy_loop(i, carry):
            sl = lax.rem(i, 2)
            _read(i, sl).wait()
            _write(i, sl).start()

            @pl.when(i + 1 < ncopy)
            def _():
                @pl.when(i >= 1)
                def _():
                    _write(i - 1, 1 - sl).wait()

                _read(i + 1, 1 - sl).start()

            return carry

        lax.fori_loop(0, ncopy, copy_loop, 0)

        @pl.when(ncopy > 0)
        def _():
            _write(ncopy - 1, lax.rem(ncopy - 1, 2)).wait()

        def row_fill(r, carry):
            pltpu.sync_copy(fill1_v, out_hbm.at[pl.ds(base + r, 1)])
            return carry

        lax.fori_loop(c, jnp.minimum(ncopy * _CH, _RPW), row_fill, 0)

        def fill_drain(i, carry):
            _fill_copy(i).wait()
            return carry

        lax.fori_loop(ncopy, _NC, fill_drain, 0)

    out = body(x2, lens.astype(jnp.int32))
    return out.reshape(_B, _S, _D)
